# Initial kernel scaffold; baseline (speedup 1.0000x reference)
import jax
import jax.numpy as jnp
from jax.experimental import pallas as pl


def kernel(x, edge_index, edge_rel, W, b, Ws1, Ws2, ln_gamma, ln_beta, w_reg, b_reg):
    raise NotImplementedError("write your pallas kernel here")


# speedup vs baseline: 15.6577x; 15.6577x over previous
import functools

import jax
import jax.numpy as jnp
from jax import lax
from jax.experimental import pallas as pl
from jax.experimental.pallas import tpu as pltpu
from jax.experimental.pallas import tpu_sc as plsc

N = 10000
E = 320000
D = 128
R = 3
DIM_A = 32

NC, NS, L = 2, 16, 16
RN = R * N
HALF = RN // 2
EPT = E // NS
CH = 80
NCH = EPT // CH

_mesh = plsc.VectorSubcoreMesh(core_axis_name="c", subcore_axis_name="s")

CNT_PAD = 30720
ZCNT = CNT_PAD // NS


@functools.partial(
    pl.kernel, mesh=_mesh,
    out_type=jax.ShapeDtypeStruct((NC * CNT_PAD,), jnp.float32),
    scratch_types=[
        pltpu.VMEM((EPT,), jnp.int32),
        pltpu.VMEM((EPT,), jnp.int32),
        pltpu.VMEM((EPT,), jnp.int32),
        pltpu.VMEM((CH,), jnp.int32),
        pltpu.VMEM((CH,), jnp.float32),
        pltpu.VMEM((ZCNT,), jnp.float32),
        pltpu.VMEM_SHARED((CNT_PAD,), jnp.float32),
    ],
)
def _sc_counts(src_hbm, dst_hbm, rel_hbm, out_hbm, sbuf, dbuf, rbuf, idx_v,
               ones_v, zbuf, acc_sh):
    c = lax.axis_index("c")
    s = lax.axis_index("s")

    def zrow(i, _):
        zbuf[pl.ds(i * L, L)] = jnp.zeros((L,), jnp.float32)
        return 0
    lax.fori_loop(0, ZCNT // L, zrow, 0)
    pltpu.sync_copy(zbuf, acc_sh.at[pl.ds(s * ZCNT, ZCNT)])

    for j in range(CH // L):
        ones_v[pl.ds(j * L, L)] = jnp.full((L,), 1.0, jnp.float32)

    base = s * EPT
    pltpu.sync_copy(src_hbm.at[pl.ds(base, EPT)], sbuf)
    pltpu.sync_copy(dst_hbm.at[pl.ds(base, EPT)], dbuf)
    pltpu.sync_copy(rel_hbm.at[pl.ds(base, EPT)], rbuf)
    plsc.subcore_barrier()

    cvec = jnp.broadcast_to(c, (L,)).astype(jnp.int32)

    def chunk(ch, _):
        off0 = ch * CH
        for j in range(CH // L):
            off = off0 + j * L
            sv = sbuf[pl.ds(off, L)]
            dv = dbuf[pl.ds(off, L)]
            nodes = sv + (dv - sv) * cvec
            rels = rbuf[pl.ds(off, L)]
            idx_v[pl.ds(j * L, L)] = rels * N + nodes
        pltpu.sync_copy(ones_v, acc_sh.at[idx_v], add=True)
        return 0

    lax.fori_loop(0, NCH, chunk, 0)
    plsc.subcore_barrier()

    @pl.when(s == 0)
    def _():
        pltpu.sync_copy(acc_sh, out_hbm.at[pl.ds(c * CNT_PAD, CNT_PAD)])


NB = 1000
NBLK = N // NB


def _prescale_body(x_ref, cnt_ref, y_ref):
    deg = cnt_ref[...]
    inv = jnp.where(deg > 0.0, lax.rsqrt(jnp.maximum(deg, 1.0)), 0.0)
    xb = x_ref[...]
    for r in range(R):
        y_ref[r] = xb * inv[:, r][:, None]


def _prescale(x, cnt_src):
    return pl.pallas_call(
        _prescale_body,
        grid=(NBLK,),
        in_specs=[
            pl.BlockSpec((NB, D), lambda i: (i, 0)),
            pl.BlockSpec((NB, R), lambda i: (i, 0)),
        ],
        out_specs=pl.BlockSpec((R, NB, D), lambda i: (0, i, 0)),
        out_shape=jax.ShapeDtypeStruct((R, N, D), jnp.float32),
    )(x, cnt_src)


KP = 3
PASS = RN // (NC * KP)
ACC_PAD = 5120
ZMSG = ACC_PAD // NS
ZROWS = 64


@functools.partial(
    pl.kernel, mesh=_mesh,
    out_type=jax.ShapeDtypeStruct((RN, D), jnp.float32),
    scratch_types=[
        pltpu.VMEM((EPT,), jnp.int32),
        pltpu.VMEM((EPT,), jnp.int32),
        pltpu.VMEM((EPT,), jnp.int32),
        pltpu.VMEM((CH,), jnp.int32),
        pltpu.VMEM((CH,), jnp.int32),
        pltpu.VMEM((CH, D), jnp.float32),
        pltpu.VMEM((ZROWS, D), jnp.float32),
        pltpu.VMEM_SHARED((ACC_PAD, D), jnp.float32),
        pltpu.SemaphoreType.DMA,
    ],
)
def _sc_messages(y_hbm, src_hbm, dst_hbm, rel_hbm, out_hbm, sbuf, dbuf, rbuf,
                 idxg, idxs, rows_v, zbuf, acc_sh, sem):
    c = lax.axis_index("c")
    s = lax.axis_index("s")

    def zrow(i, _):
        for j in range(D // L):
            zbuf[i, pl.ds(j * L, L)] = jnp.zeros((L,), jnp.float32)
        return 0
    lax.fori_loop(0, ZROWS, zrow, 0)

    base = s * EPT
    pltpu.sync_copy(src_hbm.at[pl.ds(base, EPT)], sbuf)
    pltpu.sync_copy(dst_hbm.at[pl.ds(base, EPT)], dbuf)
    pltpu.sync_copy(rel_hbm.at[pl.ds(base, EPT)], rbuf)

    for p in range(KP):
        for k in range(ZMSG // ZROWS):
            pltpu.sync_copy(zbuf, acc_sh.at[pl.ds(s * ZMSG + k * ZROWS, ZROWS)])
        plsc.subcore_barrier()

        lo = c * HALF + p * PASS

        def chunk(ch, _):
            off0 = ch * CH
            for j in range(CH // L):
                off = off0 + j * L
                sv = sbuf[pl.ds(off, L)]
                dv = dbuf[pl.ds(off, L)]
                rv = rbuf[pl.ds(off, L)]
                kg = rv * N + sv
                kd = rv * N + dv - lo
                owned = (kd >= 0) & (kd < PASS)
                neg1 = jnp.full((L,), -1, jnp.int32)
                idxg[pl.ds(j * L, L)] = jnp.where(owned, kg, neg1)
                idxs[pl.ds(j * L, L)] = jnp.where(owned, kd, neg1)
            pltpu.async_copy(
                y_hbm.at[plsc.Indices(idxg, ignored_value=-1)], rows_v, sem
            ).wait()
            pltpu.sync_copy(
                rows_v, acc_sh.at[plsc.Indices(idxs, ignored_value=-1)], add=True
            )
            return 0

        lax.fori_loop(0, NCH, chunk, 0)
        plsc.subcore_barrier()

        @pl.when(s == 0)
        def _():
            pltpu.sync_copy(acc_sh.at[pl.ds(0, PASS)], out_hbm.at[pl.ds(lo, PASS)])
        plsc.subcore_barrier()


def _dense_body(agg_ref, cnt_ref, W_ref, b_ref, Ws1_ref, Ws2_ref, g_ref,
                beta_ref, wr_ref, br_ref, out_ref, accs):
    i = pl.program_id(0)

    deg = cnt_ref[...]
    inv = jnp.where(deg > 0.0, lax.rsqrt(jnp.maximum(deg, 1.0)), 0.0)

    hs = []
    logits = []
    for r in range(R):
        ag = agg_ref[r] * inv[:, r][:, None]
        hr = jnp.dot(ag, W_ref[r], preferred_element_type=jnp.float32)
        hr = hr + b_ref[r][None, :]
        hr = jnp.where(hr > 0.0, hr, jnp.exp(jnp.minimum(hr, 0.0)) - 1.0)
        hs.append(hr)
        sg = jax.nn.sigmoid(
            jnp.dot(hr, Ws1_ref[r], preferred_element_type=jnp.float32)
        )
        logits.append(jnp.sum(sg * Ws2_ref[r][None, :], axis=-1))

    m = jnp.maximum(jnp.maximum(logits[0], logits[1]), logits[2])
    es = [jnp.exp(lg - m) for lg in logits]
    tot = es[0] + es[1] + es[2]
    comb = jnp.zeros_like(hs[0])
    for r in range(R):
        comb = comb + (es[r] / tot)[:, None] * hs[r]

    mu = jnp.mean(comb, axis=-1, keepdims=True)
    dc = comb - mu
    var = jnp.mean(dc * dc, axis=-1, keepdims=True)
    normed = dc * lax.rsqrt(var + 1e-5) * g_ref[0][None, :] + beta_ref[0][None, :]

    bsum = jnp.sum(normed, axis=0, keepdims=True)

    @pl.when(i == 0)
    def _():
        accs[...] = jnp.zeros_like(accs)

    accs[0:1, :] += bsum

    @pl.when(i == NBLK - 1)
    def _():
        pooled = accs[0:1, :] * (1.0 / N)
        val = jnp.sum(pooled * wr_ref[...], axis=1, keepdims=True)
        out_ref[...] = val + br_ref[...]


def _dense(agg, cnt_dst, W, b, Ws1, Ws2r, g, beta, wr, br):
    return pl.pallas_call(
        _dense_body,
        grid=(NBLK,),
        in_specs=[
            pl.BlockSpec((R, NB, D), lambda i: (0, i, 0)),
            pl.BlockSpec((NB, R), lambda i: (i, 0)),
            pl.BlockSpec((R, D, D), lambda i: (0, 0, 0)),
            pl.BlockSpec((R, D), lambda i: (0, 0)),
            pl.BlockSpec((R, D, DIM_A), lambda i: (0, 0, 0)),
            pl.BlockSpec((R, DIM_A), lambda i: (0, 0)),
            pl.BlockSpec((1, D), lambda i: (0, 0)),
            pl.BlockSpec((1, D), lambda i: (0, 0)),
            pl.BlockSpec((1, D), lambda i: (0, 0)),
            pl.BlockSpec((1, 1), lambda i: (0, 0)),
        ],
        out_specs=pl.BlockSpec((1, 1), lambda i: (0, 0)),
        out_shape=jax.ShapeDtypeStruct((1, 1), jnp.float32),
        scratch_shapes=[pltpu.VMEM((8, D), jnp.float32)],
    )(agg, cnt_dst, W, b, Ws1, Ws2r, g, beta, wr, br)


def kernel(x, edge_index, edge_rel, W, b, Ws1, Ws2, ln_gamma, ln_beta, w_reg,
           b_reg):
    src = edge_index[0]
    dst = edge_index[1]

    cnt = _sc_counts(src, dst, edge_rel)
    cnt_src = cnt[:RN].reshape(R, N).T
    cnt_dst = cnt[CNT_PAD:CNT_PAD + RN].reshape(R, N).T

    y = _prescale(x, cnt_src).reshape(RN, D)
    agg = _sc_messages(y, src, dst, edge_rel).reshape(R, N, D)

    pred = _dense(
        agg, cnt_dst, W, b, Ws1, Ws2[:, :, 0],
        ln_gamma.reshape(1, D), ln_beta.reshape(1, D),
        w_reg[:, 0].reshape(1, D), b_reg.reshape(1, 1),
    )
    return pred[0]


# speedup vs baseline: 22.8042x; 1.4564x over previous
import functools

import jax
import jax.numpy as jnp
from jax import lax
from jax.experimental import pallas as pl
from jax.experimental.pallas import tpu as pltpu
from jax.experimental.pallas import tpu_sc as plsc

N = 10000
E = 320000
D = 128
R = 3
DIM_A = 32

NC, NS, L = 2, 16, 16
RN = R * N
HALF = RN // 2
EPT = E // NS
CH = 80
NCH = EPT // CH

_mesh = plsc.VectorSubcoreMesh(core_axis_name="c", subcore_axis_name="s")

CNT_PAD = 30720
ZCNT = CNT_PAD // NS


@functools.partial(
    pl.kernel, mesh=_mesh,
    out_type=jax.ShapeDtypeStruct((NC * CNT_PAD,), jnp.float32),
    scratch_types=[
        pltpu.VMEM((EPT,), jnp.int32),
        pltpu.VMEM((EPT,), jnp.int32),
        pltpu.VMEM((EPT,), jnp.int32),
        pltpu.VMEM((CH,), jnp.int32),
        pltpu.VMEM((CH,), jnp.float32),
        pltpu.VMEM((ZCNT,), jnp.float32),
        pltpu.VMEM_SHARED((CNT_PAD,), jnp.float32),
    ],
)
def _sc_counts(src_hbm, dst_hbm, rel_hbm, out_hbm, sbuf, dbuf, rbuf, idx_v,
               ones_v, zbuf, acc_sh):
    c = lax.axis_index("c")
    s = lax.axis_index("s")

    def zrow(i, _):
        zbuf[pl.ds(i * L, L)] = jnp.zeros((L,), jnp.float32)
        return 0
    lax.fori_loop(0, ZCNT // L, zrow, 0)
    pltpu.sync_copy(zbuf, acc_sh.at[pl.ds(s * ZCNT, ZCNT)])

    for j in range(CH // L):
        ones_v[pl.ds(j * L, L)] = jnp.full((L,), 1.0, jnp.float32)

    base = s * EPT
    pltpu.sync_copy(src_hbm.at[pl.ds(base, EPT)], sbuf)
    pltpu.sync_copy(dst_hbm.at[pl.ds(base, EPT)], dbuf)
    pltpu.sync_copy(rel_hbm.at[pl.ds(base, EPT)], rbuf)
    plsc.subcore_barrier()

    cvec = jnp.broadcast_to(c, (L,)).astype(jnp.int32)

    def chunk(ch, _):
        off0 = ch * CH
        for j in range(CH // L):
            off = off0 + j * L
            sv = sbuf[pl.ds(off, L)]
            dv = dbuf[pl.ds(off, L)]
            nodes = sv + (dv - sv) * cvec
            rels = rbuf[pl.ds(off, L)]
            idx_v[pl.ds(j * L, L)] = rels * N + nodes
        pltpu.sync_copy(ones_v, acc_sh.at[idx_v], add=True)
        return 0

    lax.fori_loop(0, NCH, chunk, 0)
    plsc.subcore_barrier()

    @pl.when(s == 0)
    def _():
        pltpu.sync_copy(acc_sh, out_hbm.at[pl.ds(c * CNT_PAD, CNT_PAD)])


NB = 1000
NBLK = N // NB


def _prescale_body(x_ref, cnt_ref, y_ref):
    deg = cnt_ref[...]
    inv = jnp.where(deg > 0.0, lax.rsqrt(jnp.maximum(deg, 1.0)), 0.0)
    xb = x_ref[...]
    for r in range(R):
        y_ref[r] = xb * inv[:, r][:, None]


def _prescale(x, cnt_src):
    return pl.pallas_call(
        _prescale_body,
        grid=(NBLK,),
        in_specs=[
            pl.BlockSpec((NB, D), lambda i: (i, 0)),
            pl.BlockSpec((NB, R), lambda i: (i, 0)),
        ],
        out_specs=pl.BlockSpec((R, NB, D), lambda i: (0, i, 0)),
        out_shape=jax.ShapeDtypeStruct((R, N, D), jnp.float32),
    )(x, cnt_src)


KP = 3
PASS = RN // (NC * KP)
ACC_PAD = 5120
ZMSG = ACC_PAD // NS
ZROWS = 32
KSL = 5
SEC = 2
EPS = EPT // SEC
NG = EPS // (KSL * CH)


@functools.partial(
    pl.kernel, mesh=_mesh,
    out_type=jax.ShapeDtypeStruct((RN, D), jnp.float32),
    scratch_types=[
        pltpu.VMEM((EPS,), jnp.int32),
        pltpu.VMEM((EPS,), jnp.int32),
        pltpu.VMEM((EPS,), jnp.int32),
        pltpu.VMEM((KSL, CH), jnp.int32),
        pltpu.VMEM((KSL, CH), jnp.int32),
        pltpu.VMEM((KSL, CH, D), jnp.float32),
        pltpu.VMEM((ZROWS, D), jnp.float32),
        pltpu.VMEM_SHARED((ACC_PAD, D), jnp.float32),
        pltpu.SemaphoreType.DMA,
        pltpu.SemaphoreType.DMA,
    ],
)
def _sc_messages(y_hbm, src_hbm, dst_hbm, rel_hbm, out_hbm, sbuf, dbuf, rbuf,
                 idxg, idxs, rows_v, zbuf, acc_sh, semg, sems):
    c = lax.axis_index("c")
    s = lax.axis_index("s")

    def zrow(i, _):
        for j in range(D // L):
            zbuf[i, pl.ds(j * L, L)] = jnp.zeros((L,), jnp.float32)
        return 0
    lax.fori_loop(0, ZROWS, zrow, 0)

    base = s * EPT

    for p in range(KP):
        for k in range(ZMSG // ZROWS):
            pltpu.sync_copy(zbuf, acc_sh.at[pl.ds(s * ZMSG + k * ZROWS, ZROWS)])
        plsc.subcore_barrier()

        lo = c * HALF + p * PASS

        for sec in range(SEC):
            sbase = base + sec * EPS
            pltpu.sync_copy(src_hbm.at[pl.ds(sbase, EPS)], sbuf)
            pltpu.sync_copy(dst_hbm.at[pl.ds(sbase, EPS)], dbuf)
            pltpu.sync_copy(rel_hbm.at[pl.ds(sbase, EPS)], rbuf)
            _run_groups(sbuf, dbuf, rbuf, idxg, idxs, rows_v, acc_sh, y_hbm,
                        semg, sems, lo)
        plsc.subcore_barrier()

        @pl.when(s == 0)
        def _():
            pltpu.sync_copy(acc_sh.at[pl.ds(0, PASS)], out_hbm.at[pl.ds(lo, PASS)])
        plsc.subcore_barrier()


def _run_groups(sbuf, dbuf, rbuf, idxg, idxs, rows_v, acc_sh, y_hbm, semg,
                sems, lo):
        def group(g, _):
            off_g = g * (KSL * CH)
            gds = []
            for b in range(KSL):
                off0 = off_g + b * CH
                for j in range(CH // L):
                    off = off0 + j * L
                    sv = sbuf[pl.ds(off, L)]
                    dv = dbuf[pl.ds(off, L)]
                    rv = rbuf[pl.ds(off, L)]
                    kg = rv * N + sv
                    kd = rv * N + dv - lo
                    owned = (kd >= 0) & (kd < PASS)
                    neg1 = jnp.full((L,), -1, jnp.int32)
                    idxg[b, pl.ds(j * L, L)] = jnp.where(owned, kg, neg1)
                    idxs[b, pl.ds(j * L, L)] = jnp.where(owned, kd, neg1)
                gds.append(pltpu.async_copy(
                    y_hbm.at[plsc.Indices(idxg.at[b], ignored_value=-1)],
                    rows_v.at[b], semg,
                ))
            for b in range(KSL):
                gds[b].wait()
            sds = []
            for b in range(KSL):
                sds.append(pltpu.async_copy(
                    rows_v.at[b],
                    acc_sh.at[plsc.Indices(idxs.at[b], ignored_value=-1)],
                    sems, add=True,
                ))
            for b in range(KSL):
                sds[b].wait()
            return 0

        lax.fori_loop(0, NG, group, 0)


def _dense_body(agg_ref, cnt_ref, W_ref, b_ref, Ws1_ref, Ws2_ref, g_ref,
                beta_ref, wr_ref, br_ref, out_ref, accs):
    i = pl.program_id(0)

    deg = cnt_ref[...]
    inv = jnp.where(deg > 0.0, lax.rsqrt(jnp.maximum(deg, 1.0)), 0.0)

    hs = []
    logits = []
    for r in range(R):
        ag = agg_ref[r] * inv[:, r][:, None]
        hr = jnp.dot(ag, W_ref[r], preferred_element_type=jnp.float32)
        hr = hr + b_ref[r][None, :]
        hr = jnp.where(hr > 0.0, hr, jnp.exp(jnp.minimum(hr, 0.0)) - 1.0)
        hs.append(hr)
        sg = jax.nn.sigmoid(
            jnp.dot(hr, Ws1_ref[r], preferred_element_type=jnp.float32)
        )
        logits.append(jnp.sum(sg * Ws2_ref[r][None, :], axis=-1))

    m = jnp.maximum(jnp.maximum(logits[0], logits[1]), logits[2])
    es = [jnp.exp(lg - m) for lg in logits]
    tot = es[0] + es[1] + es[2]
    comb = jnp.zeros_like(hs[0])
    for r in range(R):
        comb = comb + (es[r] / tot)[:, None] * hs[r]

    mu = jnp.mean(comb, axis=-1, keepdims=True)
    dc = comb - mu
    var = jnp.mean(dc * dc, axis=-1, keepdims=True)
    normed = dc * lax.rsqrt(var + 1e-5) * g_ref[0][None, :] + beta_ref[0][None, :]

    bsum = jnp.sum(normed, axis=0, keepdims=True)

    @pl.when(i == 0)
    def _():
        accs[...] = jnp.zeros_like(accs)

    accs[0:1, :] += bsum

    @pl.when(i == NBLK - 1)
    def _():
        pooled = accs[0:1, :] * (1.0 / N)
        val = jnp.sum(pooled * wr_ref[...], axis=1, keepdims=True)
        out_ref[...] = val + br_ref[...]


def _dense(agg, cnt_dst, W, b, Ws1, Ws2r, g, beta, wr, br):
    return pl.pallas_call(
        _dense_body,
        grid=(NBLK,),
        in_specs=[
            pl.BlockSpec((R, NB, D), lambda i: (0, i, 0)),
            pl.BlockSpec((NB, R), lambda i: (i, 0)),
            pl.BlockSpec((R, D, D), lambda i: (0, 0, 0)),
            pl.BlockSpec((R, D), lambda i: (0, 0)),
            pl.BlockSpec((R, D, DIM_A), lambda i: (0, 0, 0)),
            pl.BlockSpec((R, DIM_A), lambda i: (0, 0)),
            pl.BlockSpec((1, D), lambda i: (0, 0)),
            pl.BlockSpec((1, D), lambda i: (0, 0)),
            pl.BlockSpec((1, D), lambda i: (0, 0)),
            pl.BlockSpec((1, 1), lambda i: (0, 0)),
        ],
        out_specs=pl.BlockSpec((1, 1), lambda i: (0, 0)),
        out_shape=jax.ShapeDtypeStruct((1, 1), jnp.float32),
        scratch_shapes=[pltpu.VMEM((8, D), jnp.float32)],
    )(agg, cnt_dst, W, b, Ws1, Ws2r, g, beta, wr, br)


def kernel(x, edge_index, edge_rel, W, b, Ws1, Ws2, ln_gamma, ln_beta, w_reg,
           b_reg):
    src = edge_index[0]
    dst = edge_index[1]

    cnt = _sc_counts(src, dst, edge_rel)
    cnt_src = cnt[:RN].reshape(R, N).T
    cnt_dst = cnt[CNT_PAD:CNT_PAD + RN].reshape(R, N).T

    y = _prescale(x, cnt_src).reshape(RN, D)
    agg = _sc_messages(y, src, dst, edge_rel).reshape(R, N, D)

    pred = _dense(
        agg, cnt_dst, W, b, Ws1, Ws2[:, :, 0],
        ln_gamma.reshape(1, D), ln_beta.reshape(1, D),
        w_reg[:, 0].reshape(1, D), b_reg.reshape(1, 1),
    )
    return pred[0]


# speedup vs baseline: 27.5217x; 1.2069x over previous
import functools

import jax
import jax.numpy as jnp
from jax import lax
from jax.experimental import pallas as pl
from jax.experimental.pallas import tpu as pltpu
from jax.experimental.pallas import tpu_sc as plsc

N = 10000
E = 320000
D = 128
R = 3
DIM_A = 32

NC, NS, L = 2, 16, 16
RN = R * N
HALF = RN // 2
EPT = E // NS
CH = 80
NCH = EPT // CH

_mesh = plsc.VectorSubcoreMesh(core_axis_name="c", subcore_axis_name="s")

CNT_PAD = 30720
ZCNT = CNT_PAD // NS


@functools.partial(
    pl.kernel, mesh=_mesh,
    out_type=jax.ShapeDtypeStruct((NC * CNT_PAD,), jnp.float32),
    scratch_types=[
        pltpu.VMEM((EPT,), jnp.int32),
        pltpu.VMEM((EPT,), jnp.int32),
        pltpu.VMEM((EPT,), jnp.int32),
        pltpu.VMEM((CH,), jnp.int32),
        pltpu.VMEM((CH,), jnp.float32),
        pltpu.VMEM((ZCNT,), jnp.float32),
        pltpu.VMEM_SHARED((CNT_PAD,), jnp.float32),
    ],
)
def _sc_counts(src_hbm, dst_hbm, rel_hbm, out_hbm, sbuf, dbuf, rbuf, idx_v,
               ones_v, zbuf, acc_sh):
    c = lax.axis_index("c")
    s = lax.axis_index("s")

    def zrow(i, _):
        zbuf[pl.ds(i * L, L)] = jnp.zeros((L,), jnp.float32)
        return 0
    lax.fori_loop(0, ZCNT // L, zrow, 0)
    pltpu.sync_copy(zbuf, acc_sh.at[pl.ds(s * ZCNT, ZCNT)])

    for j in range(CH // L):
        ones_v[pl.ds(j * L, L)] = jnp.full((L,), 1.0, jnp.float32)

    base = s * EPT
    pltpu.sync_copy(src_hbm.at[pl.ds(base, EPT)], sbuf)
    pltpu.sync_copy(dst_hbm.at[pl.ds(base, EPT)], dbuf)
    pltpu.sync_copy(rel_hbm.at[pl.ds(base, EPT)], rbuf)
    plsc.subcore_barrier()

    cvec = jnp.broadcast_to(c, (L,)).astype(jnp.int32)

    def chunk(ch, _):
        off0 = ch * CH
        for j in range(CH // L):
            off = off0 + j * L
            sv = sbuf[pl.ds(off, L)]
            dv = dbuf[pl.ds(off, L)]
            nodes = sv + (dv - sv) * cvec
            rels = rbuf[pl.ds(off, L)]
            idx_v[pl.ds(j * L, L)] = rels * N + nodes
        pltpu.sync_copy(ones_v, acc_sh.at[idx_v], add=True)
        return 0

    lax.fori_loop(0, NCH, chunk, 0)
    plsc.subcore_barrier()

    @pl.when(s == 0)
    def _():
        pltpu.sync_copy(acc_sh, out_hbm.at[pl.ds(c * CNT_PAD, CNT_PAD)])


NB = 1000
NBLK = N // NB


def _prescale_body(x_ref, cnt_ref, y_ref):
    deg = cnt_ref[...]
    inv = jnp.where(deg > 0.0, lax.rsqrt(jnp.maximum(deg, 1.0)), 0.0)
    xb = x_ref[...]
    for r in range(R):
        y_ref[r] = xb * inv[:, r][:, None]


def _prescale(x, cnt_src):
    return pl.pallas_call(
        _prescale_body,
        grid=(NBLK,),
        in_specs=[
            pl.BlockSpec((NB, D), lambda i: (i, 0)),
            pl.BlockSpec((NB, R), lambda i: (i, 0)),
        ],
        out_specs=pl.BlockSpec((R, NB, D), lambda i: (0, i, 0)),
        out_shape=jax.ShapeDtypeStruct((R, N, D), jnp.float32),
    )(x, cnt_src)


KP = 3
PASS = RN // (NC * KP)
ACC_PAD = 5120
ZMSG = ACC_PAD // NS
ZROWS = 8
CHM = 16
KSL = 5
GE = KSL * CHM
NG = EPT // GE


@functools.partial(
    pl.kernel, mesh=_mesh,
    out_type=jax.ShapeDtypeStruct((RN, D), jnp.float32),
    scratch_types=[
        pltpu.VMEM((EPT,), jnp.int32),
        pltpu.VMEM((EPT,), jnp.int32),
        pltpu.VMEM((EPT,), jnp.int32),
        pltpu.VMEM((2, KSL, CHM), jnp.int32),
        pltpu.VMEM((2, KSL, CHM), jnp.int32),
        pltpu.VMEM((2, KSL, CHM, D), jnp.float32),
        pltpu.VMEM((ZROWS, D), jnp.float32),
        pltpu.VMEM_SHARED((ACC_PAD, D), jnp.float32),
        pltpu.SemaphoreType.DMA,
        pltpu.SemaphoreType.DMA,
    ],
)
def _sc_messages(y_hbm, src_hbm, dst_hbm, rel_hbm, out_hbm, sbuf, dbuf, rbuf,
                 idxg, idxs, rows_v, zbuf, acc_sh, semg, sems):
    c = lax.axis_index("c")
    s = lax.axis_index("s")

    def zrow(i, _):
        for j in range(D // L):
            zbuf[i, pl.ds(j * L, L)] = jnp.zeros((L,), jnp.float32)
        return 0
    lax.fori_loop(0, ZROWS, zrow, 0)

    base = s * EPT
    pltpu.sync_copy(src_hbm.at[pl.ds(base, EPT)], sbuf)
    pltpu.sync_copy(dst_hbm.at[pl.ds(base, EPT)], dbuf)
    pltpu.sync_copy(rel_hbm.at[pl.ds(base, EPT)], rbuf)

    for p in range(KP):
        for k in range(ZMSG // ZROWS):
            pltpu.sync_copy(zbuf, acc_sh.at[pl.ds(s * ZMSG + k * ZROWS, ZROWS)])
        plsc.subcore_barrier()

        lo = c * HALF + p * PASS

        def build_fire_gathers(a, gi):
            for b in range(KSL):
                off = gi * GE + b * CHM
                sv = sbuf[pl.ds(off, L)]
                dv = dbuf[pl.ds(off, L)]
                rv = rbuf[pl.ds(off, L)]
                kg = rv * N + sv
                kd = rv * N + dv - lo
                owned = (kd >= 0) & (kd < PASS)
                neg1 = jnp.full((L,), -1, jnp.int32)
                idxg[a, b, pl.ds(0, L)] = jnp.where(owned, kg, neg1)
                idxs[a, b, pl.ds(0, L)] = jnp.where(owned, kd, neg1)
                pltpu.async_copy(
                    y_hbm.at[plsc.Indices(idxg.at[a, b], ignored_value=-1)],
                    rows_v.at[a, b], semg,
                )

        def wait_gathers(a):
            for b in range(KSL):
                pltpu.make_async_copy(
                    y_hbm.at[plsc.Indices(idxg.at[a, b], ignored_value=-1)],
                    rows_v.at[a, b], semg,
                ).wait()

        def fire_scatters(a):
            for b in range(KSL):
                pltpu.async_copy(
                    rows_v.at[a, b],
                    acc_sh.at[plsc.Indices(idxs.at[a, b], ignored_value=-1)],
                    sems, add=True,
                )

        def wait_scatters(a):
            for b in range(KSL):
                pltpu.make_async_copy(
                    rows_v.at[a, b],
                    acc_sh.at[plsc.Indices(idxs.at[a, b], ignored_value=-1)],
                    sems,
                ).wait()

        build_fire_gathers(0, 0)

        def steady(gg, _):
            g0 = gg * 2
            build_fire_gathers(1, g0 + 1)
            wait_gathers(0)
            fire_scatters(0)
            wait_gathers(1)
            fire_scatters(1)
            wait_scatters(0)
            build_fire_gathers(0, g0 + 2)
            wait_scatters(1)
            return 0

        lax.fori_loop(0, (NG - 2) // 2, steady, 0)

        build_fire_gathers(1, NG - 1)
        wait_gathers(0)
        fire_scatters(0)
        wait_gathers(1)
        fire_scatters(1)
        wait_scatters(0)
        wait_scatters(1)

        plsc.subcore_barrier()

        @pl.when(s == 0)
        def _():
            pltpu.sync_copy(acc_sh.at[pl.ds(0, PASS)], out_hbm.at[pl.ds(lo, PASS)])
        plsc.subcore_barrier()


def _dense_body(agg_ref, cnt_ref, W_ref, b_ref, Ws1_ref, Ws2_ref, g_ref,
                beta_ref, wr_ref, br_ref, out_ref, accs):
    i = pl.program_id(0)

    deg = cnt_ref[...]
    inv = jnp.where(deg > 0.0, lax.rsqrt(jnp.maximum(deg, 1.0)), 0.0)

    hs = []
    logits = []
    for r in range(R):
        ag = agg_ref[r] * inv[:, r][:, None]
        hr = jnp.dot(ag, W_ref[r], preferred_element_type=jnp.float32)
        hr = hr + b_ref[r][None, :]
        hr = jnp.where(hr > 0.0, hr, jnp.exp(jnp.minimum(hr, 0.0)) - 1.0)
        hs.append(hr)
        sg = jax.nn.sigmoid(
            jnp.dot(hr, Ws1_ref[r], preferred_element_type=jnp.float32)
        )
        logits.append(jnp.sum(sg * Ws2_ref[r][None, :], axis=-1))

    m = jnp.maximum(jnp.maximum(logits[0], logits[1]), logits[2])
    es = [jnp.exp(lg - m) for lg in logits]
    tot = es[0] + es[1] + es[2]
    comb = jnp.zeros_like(hs[0])
    for r in range(R):
        comb = comb + (es[r] / tot)[:, None] * hs[r]

    mu = jnp.mean(comb, axis=-1, keepdims=True)
    dc = comb - mu
    var = jnp.mean(dc * dc, axis=-1, keepdims=True)
    normed = dc * lax.rsqrt(var + 1e-5) * g_ref[0][None, :] + beta_ref[0][None, :]

    bsum = jnp.sum(normed, axis=0, keepdims=True)

    @pl.when(i == 0)
    def _():
        accs[...] = jnp.zeros_like(accs)

    accs[0:1, :] += bsum

    @pl.when(i == NBLK - 1)
    def _():
        pooled = accs[0:1, :] * (1.0 / N)
        val = jnp.sum(pooled * wr_ref[...], axis=1, keepdims=True)
        out_ref[...] = val + br_ref[...]


def _dense(agg, cnt_dst, W, b, Ws1, Ws2r, g, beta, wr, br):
    return pl.pallas_call(
        _dense_body,
        grid=(NBLK,),
        in_specs=[
            pl.BlockSpec((R, NB, D), lambda i: (0, i, 0)),
            pl.BlockSpec((NB, R), lambda i: (i, 0)),
            pl.BlockSpec((R, D, D), lambda i: (0, 0, 0)),
            pl.BlockSpec((R, D), lambda i: (0, 0)),
            pl.BlockSpec((R, D, DIM_A), lambda i: (0, 0, 0)),
            pl.BlockSpec((R, DIM_A), lambda i: (0, 0)),
            pl.BlockSpec((1, D), lambda i: (0, 0)),
            pl.BlockSpec((1, D), lambda i: (0, 0)),
            pl.BlockSpec((1, D), lambda i: (0, 0)),
            pl.BlockSpec((1, 1), lambda i: (0, 0)),
        ],
        out_specs=pl.BlockSpec((1, 1), lambda i: (0, 0)),
        out_shape=jax.ShapeDtypeStruct((1, 1), jnp.float32),
        scratch_shapes=[pltpu.VMEM((8, D), jnp.float32)],
    )(agg, cnt_dst, W, b, Ws1, Ws2r, g, beta, wr, br)


def kernel(x, edge_index, edge_rel, W, b, Ws1, Ws2, ln_gamma, ln_beta, w_reg,
           b_reg):
    src = edge_index[0]
    dst = edge_index[1]

    cnt = _sc_counts(src, dst, edge_rel)
    cnt_src = cnt[:RN].reshape(R, N).T
    cnt_dst = cnt[CNT_PAD:CNT_PAD + RN].reshape(R, N).T

    y = _prescale(x, cnt_src).reshape(RN, D)
    agg = _sc_messages(y, src, dst, edge_rel).reshape(R, N, D)

    pred = _dense(
        agg, cnt_dst, W, b, Ws1, Ws2[:, :, 0],
        ln_gamma.reshape(1, D), ln_beta.reshape(1, D),
        w_reg[:, 0].reshape(1, D), b_reg.reshape(1, 1),
    )
    return pred[0]


# speedup vs baseline: 28.5903x; 1.0388x over previous
import functools

import jax
import jax.numpy as jnp
from jax import lax
from jax.experimental import pallas as pl
from jax.experimental.pallas import tpu as pltpu
from jax.experimental.pallas import tpu_sc as plsc

N = 10000
E = 320000
D = 128
R = 3
DIM_A = 32

NC, NS, L = 2, 16, 16
RN = R * N
HALF = RN // 2
EPT = E // NS
CH = 80
NCH = EPT // CH

_mesh = plsc.VectorSubcoreMesh(core_axis_name="c", subcore_axis_name="s")

CNT_PAD = 30720
ZCNT = CNT_PAD // NS


@functools.partial(
    pl.kernel, mesh=_mesh,
    out_type=jax.ShapeDtypeStruct((NC * CNT_PAD,), jnp.float32),
    scratch_types=[
        pltpu.VMEM((EPT,), jnp.int32),
        pltpu.VMEM((EPT,), jnp.int32),
        pltpu.VMEM((EPT,), jnp.int32),
        pltpu.VMEM((5, CH), jnp.int32),
        pltpu.VMEM((CH,), jnp.float32),
        pltpu.VMEM((ZCNT,), jnp.float32),
        pltpu.VMEM_SHARED((CNT_PAD,), jnp.float32),
        pltpu.SemaphoreType.DMA,
    ],
)
def _sc_counts(src_hbm, dst_hbm, rel_hbm, out_hbm, sbuf, dbuf, rbuf, idx_v,
               ones_v, zbuf, acc_sh, semc):
    c = lax.axis_index("c")
    s = lax.axis_index("s")

    def zrow(i, _):
        zbuf[pl.ds(i * L, L)] = jnp.zeros((L,), jnp.float32)
        return 0
    lax.fori_loop(0, ZCNT // L, zrow, 0)
    pltpu.sync_copy(zbuf, acc_sh.at[pl.ds(s * ZCNT, ZCNT)])

    for j in range(CH // L):
        ones_v[pl.ds(j * L, L)] = jnp.full((L,), 1.0, jnp.float32)

    base = s * EPT
    pltpu.sync_copy(src_hbm.at[pl.ds(base, EPT)], sbuf)
    pltpu.sync_copy(dst_hbm.at[pl.ds(base, EPT)], dbuf)
    pltpu.sync_copy(rel_hbm.at[pl.ds(base, EPT)], rbuf)
    plsc.subcore_barrier()

    cvec = jnp.broadcast_to(c, (L,)).astype(jnp.int32)

    def group(g, _):
        for b in range(5):
            off0 = (g * 5 + b) * CH
            for j in range(CH // L):
                off = off0 + j * L
                sv = sbuf[pl.ds(off, L)]
                dv = dbuf[pl.ds(off, L)]
                nodes = sv + (dv - sv) * cvec
                rels = rbuf[pl.ds(off, L)]
                idx_v[b, pl.ds(j * L, L)] = rels * N + nodes
            pltpu.async_copy(ones_v, acc_sh.at[idx_v.at[b]], semc, add=True)
        for b in range(5):
            pltpu.make_async_copy(ones_v, acc_sh.at[idx_v.at[b]], semc).wait()
        return 0

    lax.fori_loop(0, NCH // 5, group, 0)
    plsc.subcore_barrier()

    @pl.when(s == 0)
    def _():
        pltpu.sync_copy(acc_sh, out_hbm.at[pl.ds(c * CNT_PAD, CNT_PAD)])


NB = 1000
NBLK = N // NB


def _prescale_body(x_ref, cnt_ref, y_ref):
    deg = cnt_ref[...]
    inv = jnp.where(deg > 0.0, lax.rsqrt(jnp.maximum(deg, 1.0)), 0.0)
    xb = x_ref[...]
    for r in range(R):
        y_ref[r] = xb * inv[:, r][:, None]


def _prescale(x, cnt_src):
    return pl.pallas_call(
        _prescale_body,
        grid=(NBLK,),
        in_specs=[
            pl.BlockSpec((NB, D), lambda i: (i, 0)),
            pl.BlockSpec((NB, R), lambda i: (i, 0)),
        ],
        out_specs=pl.BlockSpec((R, NB, D), lambda i: (0, i, 0)),
        out_shape=jax.ShapeDtypeStruct((R, N, D), jnp.float32),
    )(x, cnt_src)


KP = 3
PASS = RN // (NC * KP)
ACC_PAD = 5120
ZMSG = ACC_PAD // NS
ZROWS = 8
CHM = 16
KSL = 5
GE = KSL * CHM
NG = EPT // GE


@functools.partial(
    pl.kernel, mesh=_mesh,
    out_type=jax.ShapeDtypeStruct((RN, D), jnp.float32),
    scratch_types=[
        pltpu.VMEM((EPT,), jnp.int32),
        pltpu.VMEM((EPT,), jnp.int32),
        pltpu.VMEM((EPT,), jnp.int32),
        pltpu.VMEM((2, KSL, CHM), jnp.int32),
        pltpu.VMEM((2, KSL, CHM), jnp.int32),
        pltpu.VMEM((2, KSL, CHM, D), jnp.float32),
        pltpu.VMEM((ZROWS, D), jnp.float32),
        pltpu.VMEM_SHARED((ACC_PAD, D), jnp.float32),
        pltpu.SemaphoreType.DMA,
        pltpu.SemaphoreType.DMA,
    ],
)
def _sc_messages(y_hbm, src_hbm, dst_hbm, rel_hbm, out_hbm, sbuf, dbuf, rbuf,
                 idxg, idxs, rows_v, zbuf, acc_sh, semg, sems):
    c = lax.axis_index("c")
    s = lax.axis_index("s")

    def zrow(i, _):
        for j in range(D // L):
            zbuf[i, pl.ds(j * L, L)] = jnp.zeros((L,), jnp.float32)
        return 0
    lax.fori_loop(0, ZROWS, zrow, 0)

    base = s * EPT
    pltpu.sync_copy(src_hbm.at[pl.ds(base, EPT)], sbuf)
    pltpu.sync_copy(dst_hbm.at[pl.ds(base, EPT)], dbuf)
    pltpu.sync_copy(rel_hbm.at[pl.ds(base, EPT)], rbuf)

    for p in range(KP):
        for k in range(ZMSG // ZROWS):
            pltpu.async_copy(zbuf, acc_sh.at[pl.ds(s * ZMSG + k * ZROWS, ZROWS)],
                             semg)
        for k in range(ZMSG // ZROWS):
            pltpu.make_async_copy(
                zbuf, acc_sh.at[pl.ds(s * ZMSG + k * ZROWS, ZROWS)], semg
            ).wait()
        plsc.subcore_barrier()

        lo = c * HALF + p * PASS

        def build_fire_gathers(a, gi):
            for b in range(KSL):
                off = gi * GE + b * CHM
                sv = sbuf[pl.ds(off, L)]
                dv = dbuf[pl.ds(off, L)]
                rv = rbuf[pl.ds(off, L)]
                kg = rv * N + sv
                kd = rv * N + dv - lo
                owned = (kd >= 0) & (kd < PASS)
                neg1 = jnp.full((L,), -1, jnp.int32)
                idxg[a, b, pl.ds(0, L)] = jnp.where(owned, kg, neg1)
                idxs[a, b, pl.ds(0, L)] = jnp.where(owned, kd, neg1)
                pltpu.async_copy(
                    y_hbm.at[plsc.Indices(idxg.at[a, b], ignored_value=-1)],
                    rows_v.at[a, b], semg,
                )

        def wait_gathers(a):
            for b in range(KSL):
                pltpu.make_async_copy(
                    y_hbm.at[plsc.Indices(idxg.at[a, b], ignored_value=-1)],
                    rows_v.at[a, b], semg,
                ).wait()

        def fire_scatters(a):
            for b in range(KSL):
                pltpu.async_copy(
                    rows_v.at[a, b],
                    acc_sh.at[plsc.Indices(idxs.at[a, b], ignored_value=-1)],
                    sems, add=True,
                )

        def wait_scatters(a):
            for b in range(KSL):
                pltpu.make_async_copy(
                    rows_v.at[a, b],
                    acc_sh.at[plsc.Indices(idxs.at[a, b], ignored_value=-1)],
                    sems,
                ).wait()

        build_fire_gathers(0, 0)

        def steady(gg, _):
            g0 = gg * 2
            build_fire_gathers(1, g0 + 1)
            wait_gathers(0)
            fire_scatters(0)
            wait_gathers(1)
            fire_scatters(1)
            wait_scatters(0)
            build_fire_gathers(0, g0 + 2)
            wait_scatters(1)
            return 0

        lax.fori_loop(0, (NG - 2) // 2, steady, 0)

        build_fire_gathers(1, NG - 1)
        wait_gathers(0)
        fire_scatters(0)
        wait_gathers(1)
        fire_scatters(1)
        wait_scatters(0)
        wait_scatters(1)

        plsc.subcore_barrier()

        @pl.when(s == 0)
        def _():
            pltpu.sync_copy(acc_sh.at[pl.ds(0, PASS)], out_hbm.at[pl.ds(lo, PASS)])
        plsc.subcore_barrier()


def _dense_body(agg_ref, cnt_ref, W_ref, b_ref, Ws1_ref, Ws2_ref, g_ref,
                beta_ref, wr_ref, br_ref, out_ref, accs):
    i = pl.program_id(0)

    deg = cnt_ref[...]
    inv = jnp.where(deg > 0.0, lax.rsqrt(jnp.maximum(deg, 1.0)), 0.0)

    hs = []
    logits = []
    for r in range(R):
        ag = agg_ref[r] * inv[:, r][:, None]
        hr = jnp.dot(ag, W_ref[r], preferred_element_type=jnp.float32)
        hr = hr + b_ref[r][None, :]
        hr = jnp.where(hr > 0.0, hr, jnp.exp(jnp.minimum(hr, 0.0)) - 1.0)
        hs.append(hr)
        sg = jax.nn.sigmoid(
            jnp.dot(hr, Ws1_ref[r], preferred_element_type=jnp.float32)
        )
        logits.append(jnp.sum(sg * Ws2_ref[r][None, :], axis=-1))

    m = jnp.maximum(jnp.maximum(logits[0], logits[1]), logits[2])
    es = [jnp.exp(lg - m) for lg in logits]
    tot = es[0] + es[1] + es[2]
    comb = jnp.zeros_like(hs[0])
    for r in range(R):
        comb = comb + (es[r] / tot)[:, None] * hs[r]

    mu = jnp.mean(comb, axis=-1, keepdims=True)
    dc = comb - mu
    var = jnp.mean(dc * dc, axis=-1, keepdims=True)
    normed = dc * lax.rsqrt(var + 1e-5) * g_ref[0][None, :] + beta_ref[0][None, :]

    bsum = jnp.sum(normed, axis=0, keepdims=True)

    @pl.when(i == 0)
    def _():
        accs[...] = jnp.zeros_like(accs)

    accs[0:1, :] += bsum

    @pl.when(i == NBLK - 1)
    def _():
        pooled = accs[0:1, :] * (1.0 / N)
        val = jnp.sum(pooled * wr_ref[...], axis=1, keepdims=True)
        out_ref[...] = val + br_ref[...]


def _dense(agg, cnt_dst, W, b, Ws1, Ws2r, g, beta, wr, br):
    return pl.pallas_call(
        _dense_body,
        grid=(NBLK,),
        in_specs=[
            pl.BlockSpec((R, NB, D), lambda i: (0, i, 0)),
            pl.BlockSpec((NB, R), lambda i: (i, 0)),
            pl.BlockSpec((R, D, D), lambda i: (0, 0, 0)),
            pl.BlockSpec((R, D), lambda i: (0, 0)),
            pl.BlockSpec((R, D, DIM_A), lambda i: (0, 0, 0)),
            pl.BlockSpec((R, DIM_A), lambda i: (0, 0)),
            pl.BlockSpec((1, D), lambda i: (0, 0)),
            pl.BlockSpec((1, D), lambda i: (0, 0)),
            pl.BlockSpec((1, D), lambda i: (0, 0)),
            pl.BlockSpec((1, 1), lambda i: (0, 0)),
        ],
        out_specs=pl.BlockSpec((1, 1), lambda i: (0, 0)),
        out_shape=jax.ShapeDtypeStruct((1, 1), jnp.float32),
        scratch_shapes=[pltpu.VMEM((8, D), jnp.float32)],
    )(agg, cnt_dst, W, b, Ws1, Ws2r, g, beta, wr, br)


def kernel(x, edge_index, edge_rel, W, b, Ws1, Ws2, ln_gamma, ln_beta, w_reg,
           b_reg):
    src = edge_index[0]
    dst = edge_index[1]

    cnt = _sc_counts(src, dst, edge_rel)
    cnt_src = cnt[:RN].reshape(R, N).T
    cnt_dst = cnt[CNT_PAD:CNT_PAD + RN].reshape(R, N).T

    y = _prescale(x, cnt_src).reshape(RN, D)
    agg = _sc_messages(y, src, dst, edge_rel).reshape(R, N, D)

    pred = _dense(
        agg, cnt_dst, W, b, Ws1, Ws2[:, :, 0],
        ln_gamma.reshape(1, D), ln_beta.reshape(1, D),
        w_reg[:, 0].reshape(1, D), b_reg.reshape(1, 1),
    )
    return pred[0]
